# baseline (speedup 1.0000x reference)
import functools
import math

import jax
import jax.numpy as jnp
from jax import lax
from jax.experimental import pallas as pl
from jax.experimental.pallas import tpu as pltpu
from jax.experimental.pallas import tpu_sc as plsc

B, T, D, MAX_LEN = 4, 8192, 768, 8192
EPS = 1e-5
SCALE = math.sqrt(float(D))
N = B * T

_NC, _NS = 2, 16
NW = _NC * _NS

N_SC = 12288
N_TC = N - N_SC
C1, NC1 = 64, N_TC // NW // 64
C2, NC2 = 32, N_SC // NW // 32

ROWS_BLK = 512


def _ln_body(pe_ref, w_ref, b_ref, out_ref, outf_ref):
    y = pe_ref[...] * SCALE
    mu = jnp.mean(y, axis=-1, keepdims=True)
    yc = y - mu
    var = jnp.mean(yc * yc, axis=-1, keepdims=True)
    ynf = yc * lax.rsqrt(var + EPS) * w_ref[...] + b_ref[...]
    yn = ynf.astype(jnp.bfloat16)
    lo = lax.bitcast_convert_type(
        yn[:, :D // 2].astype(jnp.float32), jnp.uint32)
    hi = lax.bitcast_convert_type(
        yn[:, D // 2:].astype(jnp.float32), jnp.uint32)
    out_ref[...] = lax.bitcast_convert_type(hi | (lo >> 16), jnp.int32)
    outf_ref[...] = ynf


def _normalize_table(pe, ln_w, ln_b):
    return pl.pallas_call(
        _ln_body,
        grid=(MAX_LEN // ROWS_BLK,),
        in_specs=[
            pl.BlockSpec((ROWS_BLK, D), lambda i: (i, 0)),
            pl.BlockSpec((1, D), lambda i: (0, 0)),
            pl.BlockSpec((1, D), lambda i: (0, 0)),
        ],
        out_specs=[
            pl.BlockSpec((ROWS_BLK, D // 2), lambda i: (i, 0)),
            pl.BlockSpec((ROWS_BLK, D), lambda i: (i, 0)),
        ],
        out_shape=[
            jax.ShapeDtypeStruct((MAX_LEN, D // 2), jnp.int32),
            jax.ShapeDtypeStruct((MAX_LEN, D), jnp.float32),
        ],
    )(pe, ln_w.reshape(1, D), ln_b.reshape(1, D))


def _sc_mesh():
    return plsc.VectorSubcoreMesh(core_axis_name="c", subcore_axis_name="s",
                                  num_cores=_NC, num_subcores=_NS)


@functools.cache
def _make_gather():
    rpw = N_TC // NW

    @functools.partial(
        pl.kernel,
        out_type=jax.ShapeDtypeStruct((N_TC, D // 2), jnp.int32),
        mesh=_sc_mesh(),
        scratch_types=[
            pltpu.VMEM((NC1, C1), jnp.int32),
            pltpu.VMEM((2, C1, D // 2), jnp.int32),
            pltpu.SemaphoreType.DMA,
            pltpu.SemaphoreType.DMA,
            pltpu.SemaphoreType.DMA,
            pltpu.SemaphoreType.DMA,
        ],
    )
    def gather(table_hbm, idx_hbm, out_hbm, idx_all, gb, sg0, sg1, so0, so1):
        sg, so = (sg0, sg1), (so0, so1)
        wid = lax.axis_index("s") * _NC + lax.axis_index("c")
        base = pl.multiple_of(wid * rpw, C1)

        def start_g(b, k):
            pltpu.async_copy(table_hbm.at[idx_all.at[k]], gb.at[b], sg[b])

        def wait_g(b, k):
            pltpu.make_async_copy(table_hbm.at[idx_all.at[k]],
                                  gb.at[b], sg[b]).wait()

        def start_out(b, k):
            pltpu.async_copy(gb.at[b],
                             out_hbm.at[pl.ds(base + k * C1, C1)], so[b])

        def wait_out(b, k):
            pltpu.make_async_copy(gb.at[b],
                                  out_hbm.at[pl.ds(base + k * C1, C1)],
                                  so[b]).wait()

        pltpu.sync_copy(idx_hbm.at[wid], idx_all)
        start_g(0, 0)

        @pl.loop(0, NC1, step=2)
        def pair(k):
            @pl.when(k > 0)
            def _():
                wait_out(1, k - 1)

            start_g(1, k + 1)
            wait_g(0, k)
            start_out(0, k)

            @pl.when(k + 2 < NC1)
            def _():
                wait_out(0, k)
                start_g(0, k + 2)

            wait_g(1, k + 1)
            start_out(1, k + 1)

        wait_out(0, NC1 - 2)
        wait_out(1, NC1 - 1)

    return gather


@functools.cache
def _make_scadd():
    rpw = N_SC // NW

    @functools.partial(
        pl.kernel,
        out_type=jax.ShapeDtypeStruct((N_SC, D), jnp.float32),
        mesh=_sc_mesh(),
        scratch_types=[
            pltpu.VMEM((NC2, C2), jnp.int32),
            pltpu.VMEM((2, C2, D), jnp.float32),
            pltpu.VMEM((2, C2, D), jnp.float32),
            pltpu.SemaphoreType.DMA,
            pltpu.SemaphoreType.DMA,
            pltpu.SemaphoreType.DMA,
            pltpu.SemaphoreType.DMA,
            pltpu.SemaphoreType.DMA,
            pltpu.SemaphoreType.DMA,
        ],
    )
    def scadd(table_hbm, idx_hbm, x_hbm, out_hbm,
              idx_all, xb, gb, sg0, sg1, sx0, sx1, so0, so1):
        sg, sx, so = (sg0, sg1), (sx0, sx1), (so0, so1)
        wid = lax.axis_index("s") * _NC + lax.axis_index("c")
        xbase = pl.multiple_of(N_TC + wid * rpw, C2)
        obase = pl.multiple_of(wid * rpw, C2)

        def start_gx(b, k):
            pltpu.async_copy(table_hbm.at[idx_all.at[k]], gb.at[b], sg[b])
            pltpu.async_copy(x_hbm.at[pl.ds(xbase + k * C2, C2)],
                             xb.at[b], sx[b])

        def wait_gx(b, k):
            pltpu.make_async_copy(table_hbm.at[idx_all.at[k]],
                                  gb.at[b], sg[b]).wait()
            pltpu.make_async_copy(x_hbm.at[pl.ds(xbase + k * C2, C2)],
                                  xb.at[b], sx[b]).wait()

        def start_out(b, k):
            pltpu.async_copy(xb.at[b],
                             out_hbm.at[pl.ds(obase + k * C2, C2)], so[b])

        def wait_out(b, k):
            pltpu.make_async_copy(xb.at[b],
                                  out_hbm.at[pl.ds(obase + k * C2, C2)],
                                  so[b]).wait()

        def accum(b):
            def row(i, c):
                for j in range(D // 16):
                    sl = pl.ds(16 * j, 16)
                    plsc.addupdate(xb.at[b, i, sl], gb[b, i, sl])
                return c

            lax.fori_loop(0, C2, row, 0)

        pltpu.sync_copy(idx_hbm.at[wid], idx_all)
        start_gx(0, 0)

        @pl.loop(0, NC2, step=2)
        def pair(k):
            @pl.when(k > 0)
            def _():
                wait_out(1, k - 1)

            start_gx(1, k + 1)
            wait_gx(0, k)
            accum(0)
            start_out(0, k)
            wait_gx(1, k + 1)
            accum(1)

            @pl.when(k + 2 < NC2)
            def _():
                wait_out(0, k)
                start_gx(0, k + 2)

            start_out(1, k + 1)

        wait_out(0, NC2 - 2)
        wait_out(1, NC2 - 1)

    return scadd


ADD_BLK = 1024


def _add_body(x_ref, g_ref, out_ref):
    z = lax.bitcast_convert_type(g_ref[...], jnp.uint32)
    hi = lax.bitcast_convert_type(z & jnp.uint32(0xFFFF0000), jnp.float32)
    lo = lax.bitcast_convert_type(z << 16, jnp.float32)
    out_ref[:, :D // 2] = x_ref[:, :D // 2] + lo
    out_ref[:, D // 2:] = x_ref[:, D // 2:] + hi


def _add(x2d, gathered):
    return pl.pallas_call(
        _add_body,
        grid=(N_TC // ADD_BLK,),
        in_specs=[
            pl.BlockSpec((ADD_BLK, D), lambda i: (i, 0)),
            pl.BlockSpec((ADD_BLK, D // 2), lambda i: (i, 0)),
        ],
        out_specs=pl.BlockSpec((ADD_BLK, D), lambda i: (i, 0)),
        out_shape=jax.ShapeDtypeStruct((N, D), jnp.float32),
    )(x2d, gathered)


def kernel(x, positions, pe, ln_w, ln_b):
    table32, tablef = _normalize_table(pe, ln_w, ln_b)
    posf = positions.reshape(N).astype(jnp.int32)
    idx1 = posf[:N_TC].reshape(NW, NC1, C1)
    idx2 = posf[N_TC:].reshape(NW, NC2, C2)
    x2d = x.reshape(N, D)
    g32 = _make_gather()(table32, idx1)
    sc_rows = _make_scadd()(tablef, idx2, x2d)
    out = _add(x2d, g32)
    out = lax.dynamic_update_slice(out, sc_rows, (N_TC, 0))
    return out.reshape(B, T, D)


# speedup vs baseline: 1.1191x; 1.1191x over previous
import functools
import math

import jax
import jax.numpy as jnp
from jax import lax
from jax.experimental import pallas as pl
from jax.experimental.pallas import tpu as pltpu
from jax.experimental.pallas import tpu_sc as plsc

B, T, D, MAX_LEN = 4, 8192, 768, 8192
EPS = 1e-5
SCALE = math.sqrt(float(D))
N = B * T

_NC, _NS = 2, 16
NW = _NC * _NS

N_SC = 8192
N_TC = N - N_SC
C1, NC1 = 64, N_TC // NW // 64
C2, NC2 = 32, N_SC // NW // 32

ROWS_BLK = 512


def _ln_body(pe_ref, w_ref, b_ref, out_ref, outf_ref):
    y = pe_ref[...] * SCALE
    mu = jnp.mean(y, axis=-1, keepdims=True)
    yc = y - mu
    var = jnp.mean(yc * yc, axis=-1, keepdims=True)
    ynf = yc * lax.rsqrt(var + EPS) * w_ref[...] + b_ref[...]
    yn = ynf.astype(jnp.bfloat16)
    lo = lax.bitcast_convert_type(
        yn[:, :D // 2].astype(jnp.float32), jnp.uint32)
    hi = lax.bitcast_convert_type(
        yn[:, D // 2:].astype(jnp.float32), jnp.uint32)
    out_ref[...] = lax.bitcast_convert_type(hi | (lo >> 16), jnp.int32)
    outf_ref[...] = ynf


def _normalize_table(pe, ln_w, ln_b):
    return pl.pallas_call(
        _ln_body,
        grid=(MAX_LEN // ROWS_BLK,),
        in_specs=[
            pl.BlockSpec((ROWS_BLK, D), lambda i: (i, 0)),
            pl.BlockSpec((1, D), lambda i: (0, 0)),
            pl.BlockSpec((1, D), lambda i: (0, 0)),
        ],
        out_specs=[
            pl.BlockSpec((ROWS_BLK, D // 2), lambda i: (i, 0)),
            pl.BlockSpec((ROWS_BLK, D), lambda i: (i, 0)),
        ],
        out_shape=[
            jax.ShapeDtypeStruct((MAX_LEN, D // 2), jnp.int32),
            jax.ShapeDtypeStruct((MAX_LEN, D), jnp.float32),
        ],
    )(pe, ln_w.reshape(1, D), ln_b.reshape(1, D))


def _sc_mesh():
    return plsc.VectorSubcoreMesh(core_axis_name="c", subcore_axis_name="s",
                                  num_cores=_NC, num_subcores=_NS)


@functools.cache
def _make_gather():
    rpw = N_TC // NW

    @functools.partial(
        pl.kernel,
        out_type=jax.ShapeDtypeStruct((N_TC, D // 2), jnp.int32),
        mesh=_sc_mesh(),
        scratch_types=[
            pltpu.VMEM((NC1, C1), jnp.int32),
            pltpu.VMEM((2, C1, D // 2), jnp.int32),
            pltpu.SemaphoreType.DMA,
            pltpu.SemaphoreType.DMA,
            pltpu.SemaphoreType.DMA,
            pltpu.SemaphoreType.DMA,
        ],
    )
    def gather(table_hbm, idx_hbm, out_hbm, idx_all, gb, sg0, sg1, so0, so1):
        sg, so = (sg0, sg1), (so0, so1)
        wid = lax.axis_index("s") * _NC + lax.axis_index("c")
        base = pl.multiple_of(wid * rpw, C1)

        def start_g(b, k):
            pltpu.async_copy(table_hbm.at[idx_all.at[k]], gb.at[b], sg[b])

        def wait_g(b, k):
            pltpu.make_async_copy(table_hbm.at[idx_all.at[k]],
                                  gb.at[b], sg[b]).wait()

        def start_out(b, k):
            pltpu.async_copy(gb.at[b],
                             out_hbm.at[pl.ds(base + k * C1, C1)], so[b])

        def wait_out(b, k):
            pltpu.make_async_copy(gb.at[b],
                                  out_hbm.at[pl.ds(base + k * C1, C1)],
                                  so[b]).wait()

        pltpu.sync_copy(idx_hbm.at[wid], idx_all)
        start_g(0, 0)

        @pl.loop(0, NC1, step=2)
        def pair(k):
            @pl.when(k > 0)
            def _():
                wait_out(1, k - 1)

            start_g(1, k + 1)
            wait_g(0, k)
            start_out(0, k)

            @pl.when(k + 2 < NC1)
            def _():
                wait_out(0, k)
                start_g(0, k + 2)

            wait_g(1, k + 1)
            start_out(1, k + 1)

        wait_out(0, NC1 - 2)
        wait_out(1, NC1 - 1)

    return gather


@functools.cache
def _make_scadd():
    rpw = N_SC // NW

    @functools.partial(
        pl.kernel,
        out_type=jax.ShapeDtypeStruct((N_SC, D), jnp.float32),
        mesh=_sc_mesh(),
        scratch_types=[
            pltpu.VMEM((NC2, C2), jnp.int32),
            pltpu.VMEM((2, C2, D), jnp.float32),
            pltpu.VMEM((2, C2, D), jnp.float32),
            pltpu.SemaphoreType.DMA,
            pltpu.SemaphoreType.DMA,
            pltpu.SemaphoreType.DMA,
            pltpu.SemaphoreType.DMA,
            pltpu.SemaphoreType.DMA,
            pltpu.SemaphoreType.DMA,
        ],
    )
    def scadd(table_hbm, idx_hbm, x_hbm, out_hbm,
              idx_all, xb, gb, sg0, sg1, sx0, sx1, so0, so1):
        sg, sx, so = (sg0, sg1), (sx0, sx1), (so0, so1)
        wid = lax.axis_index("s") * _NC + lax.axis_index("c")
        xbase = pl.multiple_of(N_TC + wid * rpw, C2)
        obase = pl.multiple_of(wid * rpw, C2)

        def start_gx(b, k):
            pltpu.async_copy(table_hbm.at[idx_all.at[k]], gb.at[b], sg[b])
            pltpu.async_copy(x_hbm.at[pl.ds(xbase + k * C2, C2)],
                             xb.at[b], sx[b])

        def wait_gx(b, k):
            pltpu.make_async_copy(table_hbm.at[idx_all.at[k]],
                                  gb.at[b], sg[b]).wait()
            pltpu.make_async_copy(x_hbm.at[pl.ds(xbase + k * C2, C2)],
                                  xb.at[b], sx[b]).wait()

        def start_out(b, k):
            pltpu.async_copy(xb.at[b],
                             out_hbm.at[pl.ds(obase + k * C2, C2)], so[b])

        def wait_out(b, k):
            pltpu.make_async_copy(xb.at[b],
                                  out_hbm.at[pl.ds(obase + k * C2, C2)],
                                  so[b]).wait()

        def accum(b):
            def row(i, c):
                for j in range(D // 16):
                    sl = pl.ds(16 * j, 16)
                    plsc.addupdate(xb.at[b, i, sl], gb[b, i, sl])
                return c

            lax.fori_loop(0, C2, row, 0)

        pltpu.sync_copy(idx_hbm.at[wid], idx_all)
        start_gx(0, 0)

        @pl.loop(0, NC2, step=2)
        def pair(k):
            @pl.when(k > 0)
            def _():
                wait_out(1, k - 1)

            start_gx(1, k + 1)
            wait_gx(0, k)
            accum(0)
            start_out(0, k)
            wait_gx(1, k + 1)
            accum(1)

            @pl.when(k + 2 < NC2)
            def _():
                wait_out(0, k)
                start_gx(0, k + 2)

            start_out(1, k + 1)

        wait_out(0, NC2 - 2)
        wait_out(1, NC2 - 1)

    return scadd


ADD_BLK = 1024


def _add_body(x_ref, g_ref, out_ref):
    z = lax.bitcast_convert_type(g_ref[...], jnp.uint32)
    hi = lax.bitcast_convert_type(z & jnp.uint32(0xFFFF0000), jnp.float32)
    lo = lax.bitcast_convert_type(z << 16, jnp.float32)
    out_ref[:, :D // 2] = x_ref[:, :D // 2] + lo
    out_ref[:, D // 2:] = x_ref[:, D // 2:] + hi


def _add(x2d, gathered):
    return pl.pallas_call(
        _add_body,
        grid=(N_TC // ADD_BLK,),
        in_specs=[
            pl.BlockSpec((ADD_BLK, D), lambda i: (i, 0)),
            pl.BlockSpec((ADD_BLK, D // 2), lambda i: (i, 0)),
        ],
        out_specs=pl.BlockSpec((ADD_BLK, D), lambda i: (i, 0)),
        out_shape=jax.ShapeDtypeStruct((N, D), jnp.float32),
    )(x2d, gathered)


def _stitch_body(s_ref, oin_ref, out_ref):
    del oin_ref
    out_ref[...] = s_ref[...]


def _stitch(sc_rows, out_tc):
    return pl.pallas_call(
        _stitch_body,
        grid=(N_SC // ADD_BLK,),
        in_specs=[
            pl.BlockSpec((ADD_BLK, D), lambda i: (i, 0)),
            pl.BlockSpec((ADD_BLK, D), lambda i: (0, 0)),
        ],
        out_specs=pl.BlockSpec((ADD_BLK, D),
                               lambda i: (N_TC // ADD_BLK + i, 0)),
        out_shape=jax.ShapeDtypeStruct((N, D), jnp.float32),
        input_output_aliases={1: 0},
    )(sc_rows, out_tc)


def kernel(x, positions, pe, ln_w, ln_b):
    table32, tablef = _normalize_table(pe, ln_w, ln_b)
    posf = positions.reshape(N).astype(jnp.int32)
    idx1 = posf[:N_TC].reshape(NW, NC1, C1)
    idx2 = posf[N_TC:].reshape(NW, NC2, C2)
    x2d = x.reshape(N, D)
    g32 = _make_gather()(table32, idx1)
    sc_rows = _make_scadd()(tablef, idx2, x2d)
    out_tc = _add(x2d, g32)
    out = _stitch(sc_rows, out_tc)
    return out.reshape(B, T, D)


# speedup vs baseline: 1.2693x; 1.1342x over previous
import functools
import math

import jax
import jax.numpy as jnp
from jax import lax
from jax.experimental import pallas as pl
from jax.experimental.pallas import tpu as pltpu
from jax.experimental.pallas import tpu_sc as plsc

B, T, D, MAX_LEN = 4, 8192, 768, 8192
EPS = 1e-5
SCALE = math.sqrt(float(D))
N = B * T

ROWS_BLK = 512


def _ln_body(pe_ref, w_ref, b_ref, out_ref):
    y = pe_ref[...] * SCALE
    mu = jnp.mean(y, axis=-1, keepdims=True)
    yc = y - mu
    var = jnp.mean(yc * yc, axis=-1, keepdims=True)
    yn = (yc * lax.rsqrt(var + EPS) * w_ref[...]
          + b_ref[...]).astype(jnp.bfloat16)
    lo = lax.bitcast_convert_type(
        yn[:, :D // 2].astype(jnp.float32), jnp.uint32)
    hi = lax.bitcast_convert_type(
        yn[:, D // 2:].astype(jnp.float32), jnp.uint32)
    out_ref[...] = lax.bitcast_convert_type(hi | (lo >> 16), jnp.int32)


def _normalize_table(pe, ln_w, ln_b):
    return pl.pallas_call(
        _ln_body,
        grid=(MAX_LEN // ROWS_BLK,),
        in_specs=[
            pl.BlockSpec((ROWS_BLK, D), lambda i: (i, 0)),
            pl.BlockSpec((1, D), lambda i: (0, 0)),
            pl.BlockSpec((1, D), lambda i: (0, 0)),
        ],
        out_specs=pl.BlockSpec((ROWS_BLK, D // 2), lambda i: (i, 0)),
        out_shape=jax.ShapeDtypeStruct((MAX_LEN, D // 2), jnp.int32),
    )(pe, ln_w.reshape(1, D), ln_b.reshape(1, D))


_NC, _NS = 2, 16
NW = _NC * _NS
RPW = N // NW
CHUNK = 128
NCHUNK = RPW // CHUNK


@functools.cache
def _make_gather():
    mesh = plsc.VectorSubcoreMesh(core_axis_name="c", subcore_axis_name="s",
                                  num_cores=_NC, num_subcores=_NS)

    @functools.partial(
        pl.kernel,
        out_type=jax.ShapeDtypeStruct((N, D // 2), jnp.int32),
        mesh=mesh,
        scratch_types=[
            pltpu.VMEM((NCHUNK, CHUNK), jnp.int32),
            pltpu.VMEM((2, CHUNK, D // 2), jnp.int32),
            pltpu.SemaphoreType.DMA,
            pltpu.SemaphoreType.DMA,
            pltpu.SemaphoreType.DMA,
            pltpu.SemaphoreType.DMA,
        ],
    )
    def gather(table_hbm, idx_hbm, out_hbm, idx_all, gb, sg0, sg1, so0, so1):
        sg, so = (sg0, sg1), (so0, so1)
        wid = lax.axis_index("s") * _NC + lax.axis_index("c")
        base = pl.multiple_of(wid * RPW, CHUNK)

        def row_off(k):
            return pl.multiple_of(base + k * CHUNK, CHUNK)

        def start_g(b, k):
            pltpu.async_copy(table_hbm.at[idx_all.at[k]], gb.at[b], sg[b])

        def wait_g(b, k):
            pltpu.make_async_copy(table_hbm.at[idx_all.at[k]],
                                  gb.at[b], sg[b]).wait()

        def start_out(b, k):
            pltpu.async_copy(gb.at[b], out_hbm.at[pl.ds(row_off(k), CHUNK)],
                             so[b])

        def wait_out(b, k):
            pltpu.make_async_copy(gb.at[b],
                                  out_hbm.at[pl.ds(row_off(k), CHUNK)],
                                  so[b]).wait()

        pltpu.sync_copy(idx_hbm.at[wid], idx_all)
        start_g(0, 0)

        @pl.loop(0, NCHUNK, step=2)
        def pair(k):
            @pl.when(k > 0)
            def _():
                wait_out(1, k - 1)

            start_g(1, k + 1)
            wait_g(0, k)
            start_out(0, k)

            @pl.when(k + 2 < NCHUNK)
            def _():
                wait_out(0, k)
                start_g(0, k + 2)

            wait_g(1, k + 1)
            start_out(1, k + 1)

        wait_out(0, NCHUNK - 2)
        wait_out(1, NCHUNK - 1)

    return gather


ADD_BLK = 1024


def _add_body(x_ref, g_ref, out_ref):
    z = lax.bitcast_convert_type(g_ref[...], jnp.uint32)
    hi = lax.bitcast_convert_type(z & jnp.uint32(0xFFFF0000), jnp.float32)
    lo = lax.bitcast_convert_type(z << 16, jnp.float32)
    out_ref[:, :D // 2] = x_ref[:, :D // 2] + lo
    out_ref[:, D // 2:] = x_ref[:, D // 2:] + hi


def _add(x2d, gathered):
    return pl.pallas_call(
        _add_body,
        grid=(N // ADD_BLK,),
        in_specs=[
            pl.BlockSpec((ADD_BLK, D), lambda i: (i, 0)),
            pl.BlockSpec((ADD_BLK, D // 2), lambda i: (i, 0)),
        ],
        out_specs=pl.BlockSpec((ADD_BLK, D), lambda i: (i, 0)),
        out_shape=jax.ShapeDtypeStruct((N, D), jnp.float32),
    )(x2d, gathered)


def kernel(x, positions, pe, ln_w, ln_b):
    table32 = _normalize_table(pe, ln_w, ln_b)
    idx = positions.reshape(NW, NCHUNK, CHUNK).astype(jnp.int32)
    g32 = _make_gather()(table32, idx)
    out = _add(x.reshape(N, D), g32)
    return out.reshape(B, T, D)


# speedup vs baseline: 1.3372x; 1.0535x over previous
import functools
import math

import jax
import jax.numpy as jnp
from jax import lax
from jax.experimental import pallas as pl
from jax.experimental.pallas import tpu as pltpu
from jax.experimental.pallas import tpu_sc as plsc

B, T, D, MAX_LEN = 4, 8192, 768, 8192
EPS = 1e-5
SCALE = math.sqrt(float(D))
N = B * T

ROWS_BLK = 2048


def _ln_body(pe_ref, w_ref, b_ref, out_ref):
    y = pe_ref[...] * SCALE
    mu = jnp.mean(y, axis=-1, keepdims=True)
    yc = y - mu
    var = jnp.mean(yc * yc, axis=-1, keepdims=True)
    yn = (yc * lax.rsqrt(var + EPS) * w_ref[...]
          + b_ref[...]).astype(jnp.bfloat16)
    lo = lax.bitcast_convert_type(
        yn[:, :D // 2].astype(jnp.float32), jnp.uint32)
    hi = lax.bitcast_convert_type(
        yn[:, D // 2:].astype(jnp.float32), jnp.uint32)
    out_ref[...] = lax.bitcast_convert_type(hi | (lo >> 16), jnp.int32)


def _normalize_table(pe, ln_w, ln_b):
    return pl.pallas_call(
        _ln_body,
        grid=(MAX_LEN // ROWS_BLK,),
        in_specs=[
            pl.BlockSpec((ROWS_BLK, D), lambda i: (i, 0)),
            pl.BlockSpec((1, D), lambda i: (0, 0)),
            pl.BlockSpec((1, D), lambda i: (0, 0)),
        ],
        out_specs=pl.BlockSpec((ROWS_BLK, D // 2), lambda i: (i, 0)),
        out_shape=jax.ShapeDtypeStruct((MAX_LEN, D // 2), jnp.int32),
    )(pe, ln_w.reshape(1, D), ln_b.reshape(1, D))


_NC, _NS = 2, 16
NW = _NC * _NS
RPW = N // NW
CHUNK = 128
NCHUNK = RPW // CHUNK


@functools.cache
def _make_gather():
    mesh = plsc.VectorSubcoreMesh(core_axis_name="c", subcore_axis_name="s",
                                  num_cores=_NC, num_subcores=_NS)

    @functools.partial(
        pl.kernel,
        out_type=jax.ShapeDtypeStruct((N, D // 2), jnp.int32),
        mesh=mesh,
        scratch_types=[
            pltpu.VMEM((NCHUNK, CHUNK), jnp.int32),
            pltpu.VMEM((2, CHUNK, D // 2), jnp.int32),
            pltpu.SemaphoreType.DMA,
            pltpu.SemaphoreType.DMA,
            pltpu.SemaphoreType.DMA,
            pltpu.SemaphoreType.DMA,
        ],
    )
    def gather(table_hbm, idx_hbm, out_hbm, idx_all, gb, sg0, sg1, so0, so1):
        sg, so = (sg0, sg1), (so0, so1)
        wid = lax.axis_index("s") * _NC + lax.axis_index("c")
        base = pl.multiple_of(wid * RPW, CHUNK)

        def row_off(k):
            return pl.multiple_of(base + k * CHUNK, CHUNK)

        def start_g(b, k):
            pltpu.async_copy(table_hbm.at[idx_all.at[k]], gb.at[b], sg[b])

        def wait_g(b, k):
            pltpu.make_async_copy(table_hbm.at[idx_all.at[k]],
                                  gb.at[b], sg[b]).wait()

        def start_out(b, k):
            pltpu.async_copy(gb.at[b], out_hbm.at[pl.ds(row_off(k), CHUNK)],
                             so[b])

        def wait_out(b, k):
            pltpu.make_async_copy(gb.at[b],
                                  out_hbm.at[pl.ds(row_off(k), CHUNK)],
                                  so[b]).wait()

        pltpu.sync_copy(idx_hbm.at[wid], idx_all)
        start_g(0, 0)

        @pl.loop(0, NCHUNK, step=2)
        def pair(k):
            @pl.when(k > 0)
            def _():
                wait_out(1, k - 1)

            start_g(1, k + 1)
            wait_g(0, k)
            start_out(0, k)

            @pl.when(k + 2 < NCHUNK)
            def _():
                wait_out(0, k)
                start_g(0, k + 2)

            wait_g(1, k + 1)
            start_out(1, k + 1)

        wait_out(0, NCHUNK - 2)
        wait_out(1, NCHUNK - 1)

    return gather


ADD_BLK = 2048


def _add_body(x_ref, g_ref, out_ref):
    z = lax.bitcast_convert_type(g_ref[...], jnp.uint32)
    hi = lax.bitcast_convert_type(z & jnp.uint32(0xFFFF0000), jnp.float32)
    lo = lax.bitcast_convert_type(z << 16, jnp.float32)
    out_ref[:, :D // 2] = x_ref[:, :D // 2] + lo
    out_ref[:, D // 2:] = x_ref[:, D // 2:] + hi


def _add(x2d, gathered):
    return pl.pallas_call(
        _add_body,
        grid=(N // ADD_BLK,),
        in_specs=[
            pl.BlockSpec((ADD_BLK, D), lambda i: (i, 0)),
            pl.BlockSpec((ADD_BLK, D // 2), lambda i: (i, 0)),
        ],
        out_specs=pl.BlockSpec((ADD_BLK, D), lambda i: (i, 0)),
        out_shape=jax.ShapeDtypeStruct((N, D), jnp.float32),
    )(x2d, gathered)


def kernel(x, positions, pe, ln_w, ln_b):
    table32 = _normalize_table(pe, ln_w, ln_b)
    idx = positions.reshape(NW, NCHUNK, CHUNK).astype(jnp.int32)
    g32 = _make_gather()(table32, idx)
    out = _add(x.reshape(N, D), g32)
    return out.reshape(B, T, D)
